# baseline (speedup 1.0000x reference)
import functools

import jax
import jax.numpy as jnp
from jax import lax
from jax.experimental import pallas as pl
from jax.experimental.pallas import tpu as pltpu
from jax.experimental.pallas import tpu_sc as plsc

N = 100000
E = 1600000
D_IN = 30
H = 64

N_PAD = 100352
E_PAD = 1638400
EDGES_PER_TILE = E_PAD // 16
MB_EDGES = 2048
N_MB = EDGES_PER_TILE // MB_EDGES
BATCH = 256
CAPE = 2304

CH1 = 50048
CH2 = 20016
NCH2 = 5

BN = 512
GRID_N = N_PAD // BN


def _sc_scratch(width, ch):
    return [
        pltpu.VMEM((MB_EDGES,), jnp.int32),
        pltpu.VMEM((MB_EDGES,), jnp.int32),
        pltpu.VMEM((MB_EDGES,), jnp.int32),
        pltpu.VMEM((96,), jnp.int32),
        pltpu.VMEM((CAPE,), jnp.int32),
        pltpu.VMEM((CAPE,), jnp.int32),
        pltpu.VMEM((BATCH, width), jnp.float32),
        pltpu.VMEM((BATCH, width), jnp.float32),
        pltpu.VMEM_SHARED((ch + 16, width), jnp.float32),
        pltpu.SemaphoreType.DMA,
        pltpu.SemaphoreType.DMA,
        pltpu.SemaphoreType.DMA,
    ]


def _agg_chunk_job(src_hbm, dloc_hbm, slot_hbm, cnt_hbm, zin_hbm,
                   table_hbm, out_hbm, chunk, ch, bufs, sid):
    sm, dv, lv, cntv, srcC, dstC, rows, ssems, gsem, acc = bufs
    stripe = ch // 16
    zero = jnp.int32(0)
    one = jnp.int32(1)
    slot_lo = chunk * CAPE

    def drain(p):
        pltpu.make_async_copy(rows[p], acc.at[dstC.at[pl.ds(0, BATCH)]],
                              ssems[p]).wait()

    pltpu.sync_copy(zin_hbm, acc.at[pl.ds(sid * stripe, stripe)])
    pltpu.sync_copy(cnt_hbm.at[chunk * 16 + sid], cntv)
    plsc.subcore_barrier()
    base = sid * EDGES_PER_TILE
    dummy_s = jnp.full((16,), N, jnp.int32)
    dummy_d = jnp.full((16,), ch, jnp.int32)

    def mb_body(mb, carry):
        out0, out1 = carry
        off = base + mb * MB_EDGES
        pltpu.sync_copy(src_hbm.at[pl.ds(off, MB_EDGES)], sm)
        pltpu.sync_copy(dloc_hbm.at[pl.ds(off, MB_EDGES)], dv)
        pltpu.sync_copy(slot_hbm.at[pl.ds(off, MB_EDGES)], lv)

        @pl.when(out0 > 0)
        def _():
            drain(0)

        @pl.when(out1 > 0)
        def _():
            drain(1)

        def filt(v, c):
            s = sm[pl.ds(v * 16, 16)]
            dl = dv[pl.ds(v * 16, 16)]
            sl = lv[pl.ds(v * 16, 16)] - slot_lo
            m = (sl >= 0) & (sl < CAPE)
            pos = jnp.where(m, sl, 0)
            plsc.store_scatter(srcC, [pos], s, mask=m)
            plsc.store_scatter(dstC, [pos], dl, mask=m)
            return c
        lax.fori_loop(0, MB_EDGES // 16, filt, zero)

        cnt = cntv[pl.ds(mb, 16)][0]
        nb = (cnt + BATCH - 1) // BATCH

        lim = nb * BATCH
        def fill(q, c):
            pos = cnt + q * 16 + lax.iota(jnp.int32, 16)
            m = pos < lim
            plsc.store_scatter(srcC, [jnp.where(m, pos, 0)], dummy_s, mask=m)
            plsc.store_scatter(dstC, [jnp.where(m, pos, 0)], dummy_d, mask=m)
            return c
        lax.fori_loop(0, BATCH // 16, fill, zero)

        def batch(i, c):
            def go(p):
                @pl.when(i >= 2)
                def _():
                    drain(p)
                pltpu.async_copy(
                    table_hbm.at[srcC.at[pl.ds(i * BATCH, BATCH)]],
                    rows[p], gsem).wait()
                pltpu.async_copy(rows[p],
                                 acc.at[dstC.at[pl.ds(i * BATCH, BATCH)]],
                                 ssems[p], add=True)

            @pl.when(i % 2 == 0)
            def _():
                go(0)

            @pl.when(i % 2 == 1)
            def _():
                go(1)
            return c
        lax.fori_loop(0, nb, batch, zero)
        return (jnp.where(nb >= 1, one, zero), jnp.where(nb >= 2, one, zero))

    out0, out1 = lax.fori_loop(0, N_MB, mb_body, (zero, zero))

    @pl.when(out0 > 0)
    def _():
        drain(0)

    @pl.when(out1 > 0)
    def _():
        drain(1)
    plsc.subcore_barrier()
    pltpu.sync_copy(acc.at[pl.ds(sid * stripe, stripe)],
                    out_hbm.at[pl.ds(chunk * ch + sid * stripe, stripe)])


def _make_agg1():
    mesh = plsc.VectorSubcoreMesh(core_axis_name="c", subcore_axis_name="s")

    @functools.partial(
        pl.kernel,
        out_type=jax.ShapeDtypeStruct((N_PAD, 32), jnp.float32),
        mesh=mesh,
        compiler_params=pltpu.CompilerParams(use_tc_tiling_on_sc=False,
                                             needs_layout_passes=False),
        scratch_types=_sc_scratch(32, CH1),
    )
    def agg1(src_hbm, dloc_hbm, slot_hbm, cnt_hbm, zin_hbm, xp_hbm, a0_hbm,
             sm, dv, lv, cntv, srcC, dstC, rows0, rows1, acc_sh,
             gsem, ssem0, ssem1):
        cid = lax.axis_index("c")
        sid = lax.axis_index("s")
        bufs = (sm, dv, lv, cntv, srcC, dstC, (rows0, rows1),
                (ssem0, ssem1), gsem, acc_sh)

        @pl.when(cid == 0)
        def _():
            _agg_chunk_job(src_hbm, dloc_hbm, slot_hbm, cnt_hbm, zin_hbm,
                           xp_hbm, a0_hbm, 0, CH1, bufs, sid)

        @pl.when(cid == 1)
        def _():
            _agg_chunk_job(src_hbm, dloc_hbm, slot_hbm, cnt_hbm, zin_hbm,
                           xp_hbm, a0_hbm, 1, CH1, bufs, sid)

    return agg1


def _make_agg2():
    mesh = plsc.VectorSubcoreMesh(core_axis_name="c", subcore_axis_name="s")

    @functools.partial(
        pl.kernel,
        out_type=[jax.ShapeDtypeStruct((N_PAD, H), jnp.float32)
                  for _ in range(2)],
        mesh=mesh,
        compiler_params=pltpu.CompilerParams(use_tc_tiling_on_sc=False,
                                             needs_layout_passes=False),
        scratch_types=_sc_scratch(H, CH2),
    )
    def agg2(src_hbm, dloc_hbm, slot_hbm, cnt_hbm, zin_hbm,
             z1_hbm, z3_hbm, a1_hbm, a3_hbm,
             sm, dv, lv, cntv, srcC, dstC, rows0, rows1, acc_sh,
             gsem, ssem0, ssem1):
        cid = lax.axis_index("c")
        sid = lax.axis_index("s")
        bufs = (sm, dv, lv, cntv, srcC, dstC, (rows0, rows1),
                (ssem0, ssem1), gsem, acc_sh)

        @pl.when(cid == 0)
        def _():
            for k in range(NCH2):
                _agg_chunk_job(src_hbm, dloc_hbm, slot_hbm, cnt_hbm,
                               zin_hbm, z1_hbm, a1_hbm, k, CH2, bufs, sid)

        @pl.when(cid == 1)
        def _():
            for k in range(NCH2):
                _agg_chunk_job(src_hbm, dloc_hbm, slot_hbm, cnt_hbm,
                               zin_hbm, z3_hbm, a3_hbm, k, CH2, bufs, sid)

    return agg2


def _stage1_body(xp, a0, W1a, b1a, W1b, b1b, W3a, b3a, W3b, b3b, z1, z3):
    h = xp[...] + a0[...]
    t1 = jnp.maximum(jnp.dot(h, W1a[...],
                             preferred_element_type=jnp.float32) + b1a[...], 0.0)
    z1[...] = jnp.dot(t1, W1b[...],
                      preferred_element_type=jnp.float32) + b1b[...]
    t3 = jnp.maximum(jnp.dot(h, W3a[...],
                             preferred_element_type=jnp.float32) + b3a[...], 0.0)
    z3[...] = jnp.dot(t3, W3b[...],
                      preferred_element_type=jnp.float32) + b3b[...]


def _stage2_body(z1, a1, z3, a3, W2a, b2a, W2b, b2b, W4a, b4a, W4b, b4b,
                 zsrc, ztar):
    h1 = z1[...] + a1[...]
    t1 = jnp.maximum(jnp.dot(h1, W2a[...],
                             preferred_element_type=jnp.float32) + b2a[...], 0.0)
    zsrc[...] = jnp.dot(t1, W2b[...],
                        preferred_element_type=jnp.float32) + b2b[...]
    h3 = z3[...] + a3[...]
    t3 = jnp.maximum(jnp.dot(h3, W4a[...],
                             preferred_element_type=jnp.float32) + b4a[...], 0.0)
    ztar[...] = jnp.dot(t3, W4b[...],
                        preferred_element_type=jnp.float32) + b4b[...]


def _row_spec(w):
    return pl.BlockSpec((BN, w), lambda i: (i, 0))


def _full_spec(shape):
    return pl.BlockSpec(shape, lambda i: tuple(0 for _ in shape))


def kernel(x, edge_index, W1a, b1a, W1b, b1b, W2a, b2a, W2b, b2b,
           W3a, b3a, W3b, b3b, W4a, b4a, W4b, b4b):
    x = x.astype(jnp.float32)
    f32 = jnp.float32

    xp = jnp.pad(x, ((0, N_PAD - N), (0, 32 - D_IN)))
    src = jnp.concatenate([edge_index[0], jnp.zeros((E_PAD - E,), jnp.int32)])
    dst = jnp.concatenate([edge_index[1], jnp.full((E_PAD - E,), N, jnp.int32)])
    zin1 = jnp.zeros((CH1 // 16, 32), f32)
    zin2 = jnp.zeros((CH2 // 16, H), f32)

    dseg = dst.reshape(16, N_MB, MB_EDGES)
    U128 = jnp.triu(jnp.ones((128, 128), f32))
    U16s = jnp.triu(jnp.ones((16, 16), f32), k=1)

    def slots_for(n_chunks, chw):
        key = dseg // chw
        ar = jnp.arange(n_chunks, dtype=jnp.int32)
        oh = (key[None] == ar[:, None, None, None]).astype(f32)
        ohg = oh.reshape(n_chunks, 16, N_MB, 16, 128)
        inner = ohg @ U128
        tot = inner[..., -1]
        goff = tot @ U16s
        incl = inner + goff[..., None]
        rank = (incl.reshape(n_chunks, 16, N_MB, MB_EDGES) * oh).sum(0)
        slot = (key * CAPE + rank.astype(jnp.int32) - 1).reshape(E_PAD)
        cnt = tot.sum(-1).astype(jnp.int32)
        cnt = jnp.pad(cnt, ((0, 0), (0, 0), (0, 96 - N_MB)))
        return slot, cnt.reshape(n_chunks * 16, 96)

    slot1, cnt1 = slots_for(2, CH1)
    slot2, cnt2 = slots_for(NCH2, CH2)
    dloc1 = dst % CH1
    dloc2 = dst % CH2
    W1a_p = jnp.pad(W1a, ((0, 2), (0, 0)))
    W3a_p = jnp.pad(W3a, ((0, 2), (0, 0)))
    b1a_r, b1b_r = b1a.reshape(1, H), b1b.reshape(1, H)
    b2a_r, b2b_r = b2a.reshape(1, H), b2b.reshape(1, H)
    b3a_r, b3b_r = b3a.reshape(1, H), b3b.reshape(1, H)
    b4a_r, b4b_r = b4a.reshape(1, H), b4b.reshape(1, H)

    a0 = _make_agg1()(src, dloc1, slot1, cnt1, zin1, xp)

    stage1 = pl.pallas_call(
        _stage1_body,
        grid=(GRID_N,),
        in_specs=[_row_spec(32), _row_spec(32),
                  _full_spec((32, H)), _full_spec((1, H)),
                  _full_spec((H, H)), _full_spec((1, H)),
                  _full_spec((32, H)), _full_spec((1, H)),
                  _full_spec((H, H)), _full_spec((1, H))],
        out_specs=[_row_spec(H)] * 2,
        out_shape=[jax.ShapeDtypeStruct((N_PAD, H), f32)] * 2,
    )
    z1, z3 = stage1(xp, a0, W1a_p, b1a_r, W1b, b1b_r, W3a_p, b3a_r,
                    W3b, b3b_r)

    a1, a3 = _make_agg2()(src, dloc2, slot2, cnt2, zin2, z1, z3)

    stage2 = pl.pallas_call(
        _stage2_body,
        grid=(GRID_N,),
        in_specs=[_row_spec(H)] * 4 + [
            _full_spec((H, H)), _full_spec((1, H)),
            _full_spec((H, H)), _full_spec((1, H)),
            _full_spec((H, H)), _full_spec((1, H)),
            _full_spec((H, H)), _full_spec((1, H))],
        out_specs=[_row_spec(H)] * 2,
        out_shape=[jax.ShapeDtypeStruct((N_PAD, H), f32)] * 2,
    )
    zsrc, ztar = stage2(
        z1, a1, z3, a3, W2a, b2a_r, W2b, b2b_r, W4a, b4a_r, W4b, b4b_r)

    return (zsrc[:N], ztar[:N])


# speedup vs baseline: 2.8011x; 2.8011x over previous
import functools

import jax
import jax.numpy as jnp
from jax import lax
from jax.experimental import pallas as pl
from jax.experimental.pallas import tpu as pltpu
from jax.experimental.pallas import tpu_sc as plsc

N = 100000
E = 1600000
D_IN = 30
H = 64

N_PAD = 100352
E_ROWS = 12800
E_PAD = E_ROWS * 128
ROWS_PER_TILE = E_ROWS // 16
N_ACC = 100016
STRIPE = N_ACC // 16
MBR = 16
RB = 4
BLK_PER_MB = MBR // RB
MB_PAIRS = ROWS_PER_TILE // (2 * MBR)

BN = 512
GRID_N = N_PAD // BN

_SC_SCRATCH = [
    pltpu.VMEM((MBR, 128), jnp.int32),
    pltpu.VMEM((MBR, 128), jnp.int32),
    pltpu.VMEM((MBR, 128), jnp.int32),
    pltpu.VMEM((MBR, 128), jnp.int32),
    pltpu.VMEM((RB, 128, 16), jnp.float32),
    pltpu.VMEM((RB, 128, 16), jnp.float32),
    pltpu.VMEM_SHARED((N_ACC, 16), jnp.float32),
    pltpu.SemaphoreType.DMA,
    pltpu.SemaphoreType.DMA,
    pltpu.SemaphoreType.DMA,
]


def _agg_job(src_hbm, dst_hbm, zin_hbm, table_hbm, out_hbm, bufs, sid):
    smA, dmA, smB, dmB, rows, ssems, gsem, acc_sh = bufs

    def drain(p):
        for r in range(RB):
            pltpu.make_async_copy(rows[p].at[r], acc_sh.at[dmA.at[r]],
                                  ssems[p]).wait()

    def do_mb(sm, dm, off, first):
        pltpu.sync_copy(src_hbm.at[pl.ds(off, MBR)], sm)
        pltpu.sync_copy(dst_hbm.at[pl.ds(off, MBR)], dm)
        for blk in range(BLK_PER_MB):
            p = blk % 2
            if not (first and blk < 2):
                drain(p)
            descs = [
                pltpu.async_copy(table_hbm.at[sm.at[blk * RB + r]],
                                 rows[p].at[r], gsem)
                for r in range(RB)]
            for d in descs:
                d.wait()
            for r in range(RB):
                pltpu.async_copy(rows[p].at[r],
                                 acc_sh.at[dm.at[blk * RB + r]],
                                 ssems[p], add=True)

    pltpu.sync_copy(zin_hbm, acc_sh.at[pl.ds(sid * STRIPE, STRIPE)])
    plsc.subcore_barrier()
    base = sid * ROWS_PER_TILE

    def mb_pair(k, carry):
        off_a = base + k * (2 * MBR)

        @pl.when(k > 0)
        def _():
            drain(0)
            drain(1)

        do_mb(smA, dmA, off_a, first=True)
        do_mb(smB, dmB, off_a + MBR, first=False)
        return carry

    lax.fori_loop(0, MB_PAIRS, mb_pair, 0)
    drain(0)
    drain(1)
    plsc.subcore_barrier()
    pltpu.sync_copy(acc_sh.at[pl.ds(sid * STRIPE, STRIPE)],
                    out_hbm.at[pl.ds(sid * STRIPE, STRIPE)])


def _make_agg1():
    mesh = plsc.VectorSubcoreMesh(core_axis_name="c", subcore_axis_name="s")

    @functools.partial(
        pl.kernel,
        out_type=[jax.ShapeDtypeStruct((N_PAD, 16), jnp.float32)
                  for _ in range(2)],
        mesh=mesh,
        compiler_params=pltpu.CompilerParams(use_tc_tiling_on_sc=False),
        scratch_types=_SC_SCRATCH,
    )
    def agg1(src_hbm, dst_hbm, zin_hbm, x0_hbm, x1_hbm, a0_hbm, a1_hbm,
             smA, dmA, smB, dmB, rows0, rows1, acc_sh, gsem, ssem0, ssem1):
        cid = lax.axis_index("c")
        sid = lax.axis_index("s")
        bufs = (smA, dmA, smB, dmB, (rows0, rows1), (ssem0, ssem1),
                gsem, acc_sh)

        @pl.when(cid == 0)
        def _():
            _agg_job(src_hbm, dst_hbm, zin_hbm, x0_hbm, a0_hbm, bufs, sid)

        @pl.when(cid == 1)
        def _():
            _agg_job(src_hbm, dst_hbm, zin_hbm, x1_hbm, a1_hbm, bufs, sid)

    return agg1


def _make_agg2():
    mesh = plsc.VectorSubcoreMesh(core_axis_name="c", subcore_axis_name="s")

    @functools.partial(
        pl.kernel,
        out_type=[jax.ShapeDtypeStruct((N_PAD, 16), jnp.float32)
                  for _ in range(8)],
        mesh=mesh,
        compiler_params=pltpu.CompilerParams(use_tc_tiling_on_sc=False),
        scratch_types=_SC_SCRATCH,
    )
    def agg2(src_hbm, dst_hbm, zin_hbm,
             z10_hbm, z11_hbm, z12_hbm, z13_hbm,
             z30_hbm, z31_hbm, z32_hbm, z33_hbm,
             a10_hbm, a11_hbm, a12_hbm, a13_hbm,
             a30_hbm, a31_hbm, a32_hbm, a33_hbm,
             smA, dmA, smB, dmB, rows0, rows1, acc_sh, gsem, ssem0, ssem1):
        cid = lax.axis_index("c")
        sid = lax.axis_index("s")
        bufs = (smA, dmA, smB, dmB, (rows0, rows1), (ssem0, ssem1),
                gsem, acc_sh)

        @pl.when(cid == 0)
        def _():
            for tbl, out in ((z10_hbm, a10_hbm), (z11_hbm, a11_hbm),
                             (z12_hbm, a12_hbm), (z13_hbm, a13_hbm)):
                _agg_job(src_hbm, dst_hbm, zin_hbm, tbl, out, bufs, sid)

        @pl.when(cid == 1)
        def _():
            for tbl, out in ((z30_hbm, a30_hbm), (z31_hbm, a31_hbm),
                             (z32_hbm, a32_hbm), (z33_hbm, a33_hbm)):
                _agg_job(src_hbm, dst_hbm, zin_hbm, tbl, out, bufs, sid)

    return agg2


def _stage1_body(xp, a00, a01, W1a, b1a, W1b, b1b, W3a, b3a, W3b, b3b,
                 z10, z11, z12, z13, z30, z31, z32, z33):
    h = xp[...] + jnp.concatenate([a00[...], a01[...]], axis=1)
    t1 = jnp.maximum(jnp.dot(h, W1a[...],
                             preferred_element_type=jnp.float32) + b1a[...], 0.0)
    z1 = jnp.dot(t1, W1b[...], preferred_element_type=jnp.float32) + b1b[...]
    t3 = jnp.maximum(jnp.dot(h, W3a[...],
                             preferred_element_type=jnp.float32) + b3a[...], 0.0)
    z3 = jnp.dot(t3, W3b[...], preferred_element_type=jnp.float32) + b3b[...]
    for k, ref in enumerate((z10, z11, z12, z13)):
        ref[...] = z1[:, 16 * k:16 * (k + 1)]
    for k, ref in enumerate((z30, z31, z32, z33)):
        ref[...] = z3[:, 16 * k:16 * (k + 1)]


def _stage2_body(z10, z11, z12, z13, a10, a11, a12, a13,
                 z30, z31, z32, z33, a30, a31, a32, a33,
                 W2a, b2a, W2b, b2b, W4a, b4a, W4b, b4b,
                 zsrc, ztar):
    h1 = (jnp.concatenate([z10[...], z11[...], z12[...], z13[...]], axis=1)
          + jnp.concatenate([a10[...], a11[...], a12[...], a13[...]], axis=1))
    t1 = jnp.maximum(jnp.dot(h1, W2a[...],
                             preferred_element_type=jnp.float32) + b2a[...], 0.0)
    zsrc[...] = jnp.dot(t1, W2b[...],
                        preferred_element_type=jnp.float32) + b2b[...]
    h3 = (jnp.concatenate([z30[...], z31[...], z32[...], z33[...]], axis=1)
          + jnp.concatenate([a30[...], a31[...], a32[...], a33[...]], axis=1))
    t3 = jnp.maximum(jnp.dot(h3, W4a[...],
                             preferred_element_type=jnp.float32) + b4a[...], 0.0)
    ztar[...] = jnp.dot(t3, W4b[...],
                        preferred_element_type=jnp.float32) + b4b[...]


def _row_spec(w):
    return pl.BlockSpec((BN, w), lambda i: (i, 0))


def _full_spec(shape):
    return pl.BlockSpec(shape, lambda i: tuple(0 for _ in shape))


def kernel(x, edge_index, W1a, b1a, W1b, b1b, W2a, b2a, W2b, b2b,
           W3a, b3a, W3b, b3b, W4a, b4a, W4b, b4b):
    x = x.astype(jnp.float32)
    f32 = jnp.float32

    xp = jnp.pad(x, ((0, N_PAD - N), (0, 32 - D_IN)))
    x0, x1 = xp[:, :16], xp[:, 16:]
    src = jnp.concatenate(
        [edge_index[0], jnp.zeros((E_PAD - E,), jnp.int32)]).reshape(E_ROWS, 128)
    dst = jnp.concatenate(
        [edge_index[1], jnp.full((E_PAD - E,), N, jnp.int32)]).reshape(E_ROWS, 128)
    zin = jnp.zeros((STRIPE, 16), f32)
    W1a_p = jnp.pad(W1a, ((0, 2), (0, 0)))
    W3a_p = jnp.pad(W3a, ((0, 2), (0, 0)))
    b1a_r, b1b_r = b1a.reshape(1, H), b1b.reshape(1, H)
    b2a_r, b2b_r = b2a.reshape(1, H), b2b.reshape(1, H)
    b3a_r, b3b_r = b3a.reshape(1, H), b3b.reshape(1, H)
    b4a_r, b4b_r = b4a.reshape(1, H), b4b.reshape(1, H)

    a00, a01 = _make_agg1()(src, dst, zin, x0, x1)

    slice_shape = jax.ShapeDtypeStruct((N_PAD, 16), f32)
    stage1 = pl.pallas_call(
        _stage1_body,
        grid=(GRID_N,),
        in_specs=[_row_spec(32), _row_spec(16), _row_spec(16),
                  _full_spec((32, H)), _full_spec((1, H)),
                  _full_spec((H, H)), _full_spec((1, H)),
                  _full_spec((32, H)), _full_spec((1, H)),
                  _full_spec((H, H)), _full_spec((1, H))],
        out_specs=[_row_spec(16)] * 8,
        out_shape=[slice_shape] * 8,
    )
    z10, z11, z12, z13, z30, z31, z32, z33 = stage1(
        xp, a00, a01, W1a_p, b1a_r, W1b, b1b_r, W3a_p, b3a_r, W3b, b3b_r)

    (a10, a11, a12, a13, a30, a31, a32, a33) = _make_agg2()(
        src, dst, zin, z10, z11, z12, z13, z30, z31, z32, z33)

    out_shape = jax.ShapeDtypeStruct((N_PAD, H), f32)
    stage2 = pl.pallas_call(
        _stage2_body,
        grid=(GRID_N,),
        in_specs=[_row_spec(16)] * 16 + [
            _full_spec((H, H)), _full_spec((1, H)),
            _full_spec((H, H)), _full_spec((1, H)),
            _full_spec((H, H)), _full_spec((1, H)),
            _full_spec((H, H)), _full_spec((1, H))],
        out_specs=[_row_spec(H)] * 2,
        out_shape=[out_shape] * 2,
    )
    zsrc, ztar = stage2(
        z10, z11, z12, z13, a10, a11, a12, a13,
        z30, z31, z32, z33, a30, a31, a32, a33,
        W2a, b2a_r, W2b, b2b_r, W4a, b4a_r, W4b, b4b_r)

    return (zsrc[:N], ztar[:N])


# speedup vs baseline: 2.9299x; 1.0460x over previous
import functools

import jax
import jax.numpy as jnp
from jax import lax
from jax.experimental import pallas as pl
from jax.experimental.pallas import tpu as pltpu
from jax.experimental.pallas import tpu_sc as plsc

N = 100000
E = 1600000
D_IN = 30
H = 64

N_PAD = 100352
E_ROWS = 12800
E_PAD = E_ROWS * 128
ROWS_PER_TILE = E_ROWS // 16
N_ACC = 100016
STRIPE = N_ACC // 16
MBR = 16
RB = 4
BLK_PER_MB = MBR // RB
MB_PAIRS = ROWS_PER_TILE // (2 * MBR)

BN = 512
GRID_N = N_PAD // BN

_SC_SCRATCH = [
    pltpu.VMEM((MBR, 128), jnp.int32),
    pltpu.VMEM((MBR, 128), jnp.int32),
    pltpu.VMEM((MBR, 128), jnp.int32),
    pltpu.VMEM((MBR, 128), jnp.int32),
    pltpu.VMEM((RB, 128, 16), jnp.float32),
    pltpu.VMEM((RB, 128, 16), jnp.float32),
    pltpu.VMEM_SHARED((N_ACC, 16), jnp.float32),
    pltpu.SemaphoreType.DMA,
    pltpu.SemaphoreType.DMA,
    pltpu.SemaphoreType.DMA,
    pltpu.SemaphoreType.DMA,
    pltpu.SemaphoreType.DMA,
]


def _agg_job(src_hbm, dst_hbm, zin_hbm, table_hbm, out_hbm, bufs, sid):
    smA, dmA, smB, dmB, rows, ssems, gsem, isemA, isemB, acc_sh = bufs

    def drain(p):
        for r in range(RB):
            pltpu.make_async_copy(rows[p].at[r], acc_sh.at[dmA.at[r]],
                                  ssems[p]).wait()

    def do_mb(sm, dm, first):
        for blk in range(BLK_PER_MB):
            p = blk % 2
            if not (first and blk < 2):
                drain(p)
            descs = [
                pltpu.async_copy(table_hbm.at[sm.at[blk * RB + r]],
                                 rows[p].at[r], gsem)
                for r in range(RB)]
            for d in descs:
                d.wait()
            for r in range(RB):
                pltpu.async_copy(rows[p].at[r],
                                 acc_sh.at[dm.at[blk * RB + r]],
                                 ssems[p], add=True)

    pltpu.sync_copy(zin_hbm, acc_sh.at[pl.ds(sid * STRIPE, STRIPE)])
    plsc.subcore_barrier()
    base = sid * ROWS_PER_TILE

    def mb_pair(k, carry):
        off_a = base + k * (2 * MBR)

        @pl.when(k > 0)
        def _():
            drain(0)
            drain(1)

        dA = [pltpu.async_copy(src_hbm.at[pl.ds(off_a, MBR)], smA, isemA),
              pltpu.async_copy(dst_hbm.at[pl.ds(off_a, MBR)], dmA, isemA)]
        dB = [pltpu.async_copy(src_hbm.at[pl.ds(off_a + MBR, MBR)], smB,
                               isemB),
              pltpu.async_copy(dst_hbm.at[pl.ds(off_a + MBR, MBR)], dmB,
                               isemB)]
        for d in dA:
            d.wait()
        do_mb(smA, dmA, first=True)
        for d in dB:
            d.wait()
        do_mb(smB, dmB, first=False)
        return carry

    lax.fori_loop(0, MB_PAIRS, mb_pair, 0)
    drain(0)
    drain(1)
    plsc.subcore_barrier()
    pltpu.sync_copy(acc_sh.at[pl.ds(sid * STRIPE, STRIPE)],
                    out_hbm.at[pl.ds(sid * STRIPE, STRIPE)])


def _make_agg1():
    mesh = plsc.VectorSubcoreMesh(core_axis_name="c", subcore_axis_name="s")

    @functools.partial(
        pl.kernel,
        out_type=[jax.ShapeDtypeStruct((N_PAD, 16), jnp.float32)
                  for _ in range(2)],
        mesh=mesh,
        compiler_params=pltpu.CompilerParams(use_tc_tiling_on_sc=False),
        scratch_types=_SC_SCRATCH,
    )
    def agg1(src_hbm, dst_hbm, zin_hbm, x0_hbm, x1_hbm, a0_hbm, a1_hbm,
             smA, dmA, smB, dmB, rows0, rows1, acc_sh, gsem, ssem0, ssem1,
             isemA, isemB):
        cid = lax.axis_index("c")
        sid = lax.axis_index("s")
        bufs = (smA, dmA, smB, dmB, (rows0, rows1), (ssem0, ssem1),
                gsem, isemA, isemB, acc_sh)

        @pl.when(cid == 0)
        def _():
            _agg_job(src_hbm, dst_hbm, zin_hbm, x0_hbm, a0_hbm, bufs, sid)

        @pl.when(cid == 1)
        def _():
            _agg_job(src_hbm, dst_hbm, zin_hbm, x1_hbm, a1_hbm, bufs, sid)

    return agg1


def _make_agg2():
    mesh = plsc.VectorSubcoreMesh(core_axis_name="c", subcore_axis_name="s")

    @functools.partial(
        pl.kernel,
        out_type=[jax.ShapeDtypeStruct((N_PAD, 16), jnp.float32)
                  for _ in range(8)],
        mesh=mesh,
        compiler_params=pltpu.CompilerParams(use_tc_tiling_on_sc=False),
        scratch_types=_SC_SCRATCH,
    )
    def agg2(src_hbm, dst_hbm, zin_hbm,
             z10_hbm, z11_hbm, z12_hbm, z13_hbm,
             z30_hbm, z31_hbm, z32_hbm, z33_hbm,
             a10_hbm, a11_hbm, a12_hbm, a13_hbm,
             a30_hbm, a31_hbm, a32_hbm, a33_hbm,
             smA, dmA, smB, dmB, rows0, rows1, acc_sh, gsem, ssem0, ssem1,
             isemA, isemB):
        cid = lax.axis_index("c")
        sid = lax.axis_index("s")
        bufs = (smA, dmA, smB, dmB, (rows0, rows1), (ssem0, ssem1),
                gsem, isemA, isemB, acc_sh)

        @pl.when(cid == 0)
        def _():
            for tbl, out in ((z10_hbm, a10_hbm), (z11_hbm, a11_hbm),
                             (z12_hbm, a12_hbm), (z13_hbm, a13_hbm)):
                _agg_job(src_hbm, dst_hbm, zin_hbm, tbl, out, bufs, sid)

        @pl.when(cid == 1)
        def _():
            for tbl, out in ((z30_hbm, a30_hbm), (z31_hbm, a31_hbm),
                             (z32_hbm, a32_hbm), (z33_hbm, a33_hbm)):
                _agg_job(src_hbm, dst_hbm, zin_hbm, tbl, out, bufs, sid)

    return agg2


def _stage1_body(xp, a00, a01, W1a, b1a, W1b, b1b, W3a, b3a, W3b, b3b,
                 z10, z11, z12, z13, z30, z31, z32, z33):
    h = xp[...] + jnp.concatenate([a00[...], a01[...]], axis=1)
    t1 = jnp.maximum(jnp.dot(h, W1a[...],
                             preferred_element_type=jnp.float32) + b1a[...], 0.0)
    z1 = jnp.dot(t1, W1b[...], preferred_element_type=jnp.float32) + b1b[...]
    t3 = jnp.maximum(jnp.dot(h, W3a[...],
                             preferred_element_type=jnp.float32) + b3a[...], 0.0)
    z3 = jnp.dot(t3, W3b[...], preferred_element_type=jnp.float32) + b3b[...]
    for k, ref in enumerate((z10, z11, z12, z13)):
        ref[...] = z1[:, 16 * k:16 * (k + 1)]
    for k, ref in enumerate((z30, z31, z32, z33)):
        ref[...] = z3[:, 16 * k:16 * (k + 1)]


def _stage2_body(z10, z11, z12, z13, a10, a11, a12, a13,
                 z30, z31, z32, z33, a30, a31, a32, a33,
                 W2a, b2a, W2b, b2b, W4a, b4a, W4b, b4b,
                 zsrc, ztar):
    h1 = (jnp.concatenate([z10[...], z11[...], z12[...], z13[...]], axis=1)
          + jnp.concatenate([a10[...], a11[...], a12[...], a13[...]], axis=1))
    t1 = jnp.maximum(jnp.dot(h1, W2a[...],
                             preferred_element_type=jnp.float32) + b2a[...], 0.0)
    zsrc[...] = jnp.dot(t1, W2b[...],
                        preferred_element_type=jnp.float32) + b2b[...]
    h3 = (jnp.concatenate([z30[...], z31[...], z32[...], z33[...]], axis=1)
          + jnp.concatenate([a30[...], a31[...], a32[...], a33[...]], axis=1))
    t3 = jnp.maximum(jnp.dot(h3, W4a[...],
                             preferred_element_type=jnp.float32) + b4a[...], 0.0)
    ztar[...] = jnp.dot(t3, W4b[...],
                        preferred_element_type=jnp.float32) + b4b[...]


def _row_spec(w):
    return pl.BlockSpec((BN, w), lambda i: (i, 0))


def _full_spec(shape):
    return pl.BlockSpec(shape, lambda i: tuple(0 for _ in shape))


def kernel(x, edge_index, W1a, b1a, W1b, b1b, W2a, b2a, W2b, b2b,
           W3a, b3a, W3b, b3b, W4a, b4a, W4b, b4b):
    x = x.astype(jnp.float32)
    f32 = jnp.float32

    xp = jnp.pad(x, ((0, N_PAD - N), (0, 32 - D_IN)))
    x0, x1 = xp[:, :16], xp[:, 16:]
    src = jnp.concatenate(
        [edge_index[0], jnp.zeros((E_PAD - E,), jnp.int32)]).reshape(E_ROWS, 128)
    dst = jnp.concatenate(
        [edge_index[1], jnp.full((E_PAD - E,), N, jnp.int32)]).reshape(E_ROWS, 128)
    zin = jnp.zeros((STRIPE, 16), f32)
    W1a_p = jnp.pad(W1a, ((0, 2), (0, 0)))
    W3a_p = jnp.pad(W3a, ((0, 2), (0, 0)))
    b1a_r, b1b_r = b1a.reshape(1, H), b1b.reshape(1, H)
    b2a_r, b2b_r = b2a.reshape(1, H), b2b.reshape(1, H)
    b3a_r, b3b_r = b3a.reshape(1, H), b3b.reshape(1, H)
    b4a_r, b4b_r = b4a.reshape(1, H), b4b.reshape(1, H)

    a00, a01 = _make_agg1()(src, dst, zin, x0, x1)

    slice_shape = jax.ShapeDtypeStruct((N_PAD, 16), f32)
    stage1 = pl.pallas_call(
        _stage1_body,
        grid=(GRID_N,),
        in_specs=[_row_spec(32), _row_spec(16), _row_spec(16),
                  _full_spec((32, H)), _full_spec((1, H)),
                  _full_spec((H, H)), _full_spec((1, H)),
                  _full_spec((32, H)), _full_spec((1, H)),
                  _full_spec((H, H)), _full_spec((1, H))],
        out_specs=[_row_spec(16)] * 8,
        out_shape=[slice_shape] * 8,
    )
    z10, z11, z12, z13, z30, z31, z32, z33 = stage1(
        xp, a00, a01, W1a_p, b1a_r, W1b, b1b_r, W3a_p, b3a_r, W3b, b3b_r)

    (a10, a11, a12, a13, a30, a31, a32, a33) = _make_agg2()(
        src, dst, zin, z10, z11, z12, z13, z30, z31, z32, z33)

    out_shape = jax.ShapeDtypeStruct((N_PAD, H), f32)
    stage2 = pl.pallas_call(
        _stage2_body,
        grid=(GRID_N,),
        in_specs=[_row_spec(16)] * 16 + [
            _full_spec((H, H)), _full_spec((1, H)),
            _full_spec((H, H)), _full_spec((1, H)),
            _full_spec((H, H)), _full_spec((1, H)),
            _full_spec((H, H)), _full_spec((1, H))],
        out_specs=[_row_spec(H)] * 2,
        out_shape=[out_shape] * 2,
    )
    zsrc, ztar = stage2(
        z10, z11, z12, z13, a10, a11, a12, a13,
        z30, z31, z32, z33, a30, a31, a32, a33,
        W2a, b2a_r, W2b, b2b_r, W4a, b4a_r, W4b, b4b_r)

    return (zsrc[:N], ztar[:N])
